# baseline (speedup 1.0000x reference)
import functools

import jax
import jax.numpy as jnp
from jax import lax
from jax.experimental import pallas as pl
from jax.experimental.pallas import tpu as pltpu
from jax.experimental.pallas import tpu_sc as plsc

_NUM_OBJS = 1000
_NUM_RELS = 64
_BATCH = 16384
_V = _NUM_OBJS * _NUM_OBJS
_L = 16
_BLK = 128
_NBLK = (_V + _BLK - 1) // _BLK
_TAILB = _NBLK - 1
_TAIL0 = _TAILB * _BLK
_RING = 6
_CAP = 16
_CHUNK = 2048


@functools.lru_cache(maxsize=None)
def _build(num_cores: int, num_subcores: int):
    nw = num_cores * num_subcores
    bpb = (_NBLK + nw - 1) // nw
    nvis = (bpb + _RING - 1) // _RING * _RING
    mesh = plsc.VectorSubcoreMesh(
        core_axis_name="c", subcore_axis_name="s",
        num_cores=num_cores, num_subcores=num_subcores)

    slot_types = [pltpu.VMEM((_NUM_RELS, _BLK), jnp.float32)
                  for _ in range(_RING)]
    lsem_types = [pltpu.SemaphoreType.DMA for _ in range(_RING)]
    esem_types = [pltpu.SemaphoreType.DMA for _ in range(_RING)]

    @functools.partial(
        pl.kernel,
        out_type=(jax.ShapeDtypeStruct((_BATCH, _NUM_RELS), jnp.float32),
                  jax.ShapeDtypeStruct((nw, _NUM_RELS), jnp.float32)),
        mesh=mesh,
        scratch_types=[
            pltpu.VMEM((_CHUNK,), jnp.int32),
            pltpu.VMEM((_CHUNK,), jnp.int32),
            pltpu.VMEM((bpb, 2 * _CAP), jnp.int32),
            pltpu.VMEM((bpb, 2 * _CAP), jnp.int32),
            pltpu.VMEM((_NUM_RELS // 2, _BLK), jnp.float32),
            pltpu.VMEM((_NUM_RELS, _BLK), jnp.float32),
            pltpu.VMEM((_NUM_RELS,), jnp.int32),
            pltpu.SMEM((bpb + 8,), jnp.int32),
            *slot_types,
            *lsem_types,
            *esem_types,
            pltpu.SemaphoreType.DMA,
        ],
    )
    def k(l0_hbm, l1_hbm, table_hbm, tail_hbm, out_hbm, scrap_hbm,
          l0_v, l1_v, bkt_m, bkt_p, tail_v, spare_v, dr_v, smem, *rest):
        slots = rest[:_RING]
        lsems = rest[_RING:2 * _RING]
        esems = rest[2 * _RING:3 * _RING]
        ovf_sem = rest[3 * _RING]
        wid = lax.axis_index("s") * num_cores + lax.axis_index("c")
        wbase = wid * bpb
        lo = wbase * _BLK
        span = bpb * _BLK
        lane = lax.iota(jnp.int32, _L)

        def issue_load(local_i, slot, sem):
            b = jnp.minimum(wbase + local_i, _TAILB - 1)
            off = pl.multiple_of(b * _BLK, _BLK)
            pltpu.async_copy(table_hbm.at[:, pl.ds(off, _BLK)], slot, sem)

        for s in range(_RING // 2):
            issue_load(s, slots[s], lsems[s])
        pltpu.sync_copy(tail_hbm, tail_v)
        def zcnt(i, _):
            smem[i] = 0
            return 0
        lax.fori_loop(0, bpb, zcnt, 0)

        def out_hit(p, r_m, block_b, resident_ref, sem):
            @pl.when(block_b == _TAILB)
            def _():
                pltpu.async_copy(
                    tail_v.at[r_m >> 1, pl.ds((r_m & 1) * _NUM_RELS,
                                              _NUM_RELS)],
                    out_hbm.at[p], sem)
            @pl.when(block_b != _TAILB)
            def _():
                pltpu.async_copy(resident_ref.at[:, r_m], out_hbm.at[p], sem)

        def det_chunk(cb, _):
            pltpu.sync_copy(l0_hbm.at[pl.ds(cb * _CHUNK, _CHUNK)], l0_v)
            pltpu.sync_copy(l1_hbm.at[pl.ds(cb * _CHUNK, _CHUNK)], l1_v)

            def det_group(g, _):
                f = (l0_v[pl.ds(g * _L, _L)] * _NUM_OBJS
                     + l1_v[pl.ds(g * _L, _L)])
                rel = f - lo
                mi = jnp.where((rel >= 0) & (rel < span), 1, 0)
                t1 = mi + mi.at[lane ^ 8].get(mode="promise_in_bounds")
                t2 = t1 + t1.at[lane ^ 4].get(mode="promise_in_bounds")
                t3 = t2 + t2.at[lane ^ 2].get(mode="promise_in_bounds")
                t4 = t3 + t3.at[lane ^ 1].get(mode="promise_in_bounds")
                @pl.when(t4[0] > 0)
                def _():
                    for j in range(_L):
                        @pl.when(mi[j] > 0)
                        def _():
                            r = f[j]
                            i = (r >> 7) - wbase
                            m = r & (_BLK - 1)
                            p = cb * _CHUNK + g * _L + j
                            c = smem[i]
                            @pl.when(c < _CAP)
                            def _():
                                bkt_m[i, pl.ds(c, _L)] = lane * 0 + m
                                bkt_p[i, pl.ds(c, _L)] = lane * 0 + p
                            @pl.when(c >= _CAP)
                            def _():
                                b = r >> 7
                                @pl.when(b != _TAILB)
                                def _():
                                    boff = pl.multiple_of(
                                        jnp.minimum(b, _TAILB - 1) * _BLK,
                                        _BLK)
                                    pltpu.sync_copy(
                                        table_hbm.at[:, pl.ds(boff, _BLK)],
                                        spare_v)
                                out_hit(p, m, b, spare_v, ovf_sem)
                                pltpu.make_async_copy(
                                    l0_hbm.at[pl.ds(0, _NUM_RELS)],
                                    dr_v, ovf_sem).wait()
                            smem[i] = c + 1
                return 0
            lax.fori_loop(0, _CHUNK // _L, det_group, 0)
            return 0
        lax.fori_loop(0, _BATCH // _CHUNK, det_chunk, 0)

        def active(local_i):
            return (local_i >= 0) & (local_i < bpb) & (wbase + local_i < _NBLK)

        def wave(t, _):
            for s in range(_RING):
                i = t * _RING + s
                j = (s + _RING // 2) % _RING
                @pl.when(active(i - _RING // 2))
                def _():
                    pltpu.make_async_copy(
                        table_hbm.at[pl.ds(0, 8), pl.ds(0, _BLK)],
                        slots[j].at[pl.ds(0, 8), :], esems[j]).wait()
                @pl.when(i + _RING // 2 < nvis)
                def _():
                    issue_load(i + _RING // 2, slots[j], lsems[j])
                @pl.when(i < nvis)
                def _():
                    pltpu.make_async_copy(
                        table_hbm.at[:, pl.ds(0, _BLK)],
                        slots[s], lsems[s]).wait()
                @pl.when(active(i))
                def _():
                    b = wbase + i
                    cnt = jnp.minimum(smem[i], _CAP)
                    def ext(e, _):
                        m = bkt_m[i, pl.ds(e, _L)][0]
                        p = bkt_p[i, pl.ds(e, _L)][0]
                        out_hit(p, m, b, slots[s], esems[s])
                        return 0
                    lax.fori_loop(0, cnt, ext, 0)
                    def pad(e, _):
                        pltpu.async_copy(slots[s].at[:, 0],
                                         scrap_hbm.at[wid], esems[s])
                        return 0
                    lax.fori_loop(cnt, _CAP, pad, 0)
            return 0
        lax.fori_loop(0, nvis // _RING, wave, 0)
        for s in range(_RING):
            i_last = nvis - _RING + s
            @pl.when(active(i_last) & (i_last + _RING // 2 >= nvis))
            def _():
                pltpu.make_async_copy(
                    table_hbm.at[pl.ds(0, 8), pl.ds(0, _BLK)],
                    slots[s].at[pl.ds(0, 8), :], esems[s]).wait()

    return k


def kernel(labels, obj_baseline):
    info = plsc.get_sparse_core_info()
    k = _build(info.num_cores, info.num_subcores)
    tail = obj_baseline[_TAIL0:].reshape(_NUM_RELS // 2, _BLK)
    out, _scrap = k(labels[:, 0], labels[:, 1], obj_baseline.T, tail)
    return out
